# Initial kernel scaffold; baseline (speedup 1.0000x reference)
import jax
import jax.numpy as jnp
from jax.experimental import pallas as pl


def kernel(x, edge_index, W1, b1, W2, b2):
    raise NotImplementedError("write your pallas kernel here")


# speedup vs baseline: 37.1446x; 37.1446x over previous
import functools

import jax
import jax.numpy as jnp
from jax import lax
from jax.experimental import pallas as pl
from jax.experimental.pallas import tpu as pltpu
from jax.experimental.pallas import tpu_sc as plsc

N_NODES = 10000
D_IN = 128
D = 16

NC = 2
NS = 16
NW = NC * NS

N_PAD = 10240
CHUNK = N_PAD // NS

EW = 128
E_EDGES = 320000
ROWS_PER_TILE = 80
PER_TILE = ROWS_PER_TILE * EW
E_PAD = NW * PER_TILE
E_ROWS = E_PAD // EW

_mesh = plsc.VectorSubcoreMesh(core_axis_name="c", subcore_axis_name="s")


@functools.partial(
    pl.kernel,
    out_type=jax.ShapeDtypeStruct((NC, N_PAD), jnp.float32),
    mesh=_mesh,
    scratch_types=[
        pltpu.VMEM((ROWS_PER_TILE, EW), jnp.int32),
        pltpu.VMEM((EW,), jnp.float32),
        pltpu.VMEM_SHARED((N_PAD,), jnp.float32),
    ],
)
def _sc_degree(dst_hbm, zeros1_hbm, out_hbm, didx_v, ones_v, deg_sh):
    c = lax.axis_index("c")
    s = lax.axis_index("s")
    w = c * NS + s
    pltpu.sync_copy(zeros1_hbm.at[pl.ds(s * CHUNK, CHUNK)],
                    deg_sh.at[pl.ds(s * CHUNK, CHUNK)])
    pltpu.sync_copy(dst_hbm.at[pl.ds(w * ROWS_PER_TILE, ROWS_PER_TILE)], didx_v)
    for i in range(EW // 16):
        ones_v[pl.ds(i * 16, 16)] = jnp.ones((16,), jnp.float32)
    plsc.subcore_barrier()

    def body(j, carry):
        pltpu.sync_copy(ones_v, deg_sh.at[didx_v.at[j]], add=True)
        return carry

    lax.fori_loop(0, ROWS_PER_TILE, body, 0)
    plsc.subcore_barrier()
    pltpu.sync_copy(deg_sh.at[pl.ds(s * CHUNK, CHUNK)],
                    out_hbm.at[c, pl.ds(s * CHUNK, CHUNK)])


@functools.partial(
    pl.kernel,
    out_type=jax.ShapeDtypeStruct((NC, N_PAD, D), jnp.float32),
    mesh=_mesh,
    scratch_types=[
        pltpu.VMEM((ROWS_PER_TILE, EW), jnp.int32),
        pltpu.VMEM((ROWS_PER_TILE, EW), jnp.int32),
        pltpu.VMEM((EW, D), jnp.float32),
        pltpu.VMEM_SHARED((N_PAD, D), jnp.float32),
        pltpu.SemaphoreType.DMA,
    ],
    compiler_params=pltpu.CompilerParams(use_tc_tiling_on_sc=False),
)
def _sc_aggregate(g_hbm, src_hbm, dst_hbm, zeros2_hbm, out_hbm,
                  sidx_v, didx_v, rows_v, acc_sh, sem):
    c = lax.axis_index("c")
    s = lax.axis_index("s")
    w = c * NS + s
    pltpu.sync_copy(zeros2_hbm.at[pl.ds(s * CHUNK, CHUNK)],
                    acc_sh.at[pl.ds(s * CHUNK, CHUNK)])
    pltpu.sync_copy(src_hbm.at[pl.ds(w * ROWS_PER_TILE, ROWS_PER_TILE)], sidx_v)
    pltpu.sync_copy(dst_hbm.at[pl.ds(w * ROWS_PER_TILE, ROWS_PER_TILE)], didx_v)
    plsc.subcore_barrier()

    def body(j, carry):
        pltpu.async_copy(g_hbm.at[sidx_v.at[j]], rows_v, sem).wait()
        pltpu.sync_copy(rows_v, acc_sh.at[didx_v.at[j]], add=True)
        return carry

    lax.fori_loop(0, ROWS_PER_TILE, body, 0)
    plsc.subcore_barrier()
    pltpu.sync_copy(acc_sh.at[pl.ds(s * CHUNK, CHUNK)],
                    out_hbm.at[c, pl.ds(s * CHUNK, CHUNK)])


def _dinv(degp_ref):
    deg = degp_ref[0, :] + degp_ref[1, :] + 1.0
    return lax.rsqrt(deg)


def _tc1_body(x_ref, w1_ref, degp_ref, g1_ref):
    dinv = _dinv(degp_ref)
    h = jnp.dot(x_ref[...], w1_ref[...], preferred_element_type=jnp.float32)
    g1_ref[...] = h * dinv[:, None]


def _tc2_body(p_ref, g1_ref, degp_ref, b1_ref, w2_ref, g2_ref):
    dinv = _dinv(degp_ref)
    s1 = p_ref[0] + p_ref[1] + g1_ref[...]
    a1 = s1 * dinv[:, None] + b1_ref[...][None, :]
    h1 = jnp.maximum(a1, 0.0)
    h2 = jnp.dot(h1, w2_ref[...], preferred_element_type=jnp.float32)
    g2_ref[...] = h2 * dinv[:, None]


def _tc3_body(p_ref, g2_ref, degp_ref, b2_ref, out_ref):
    dinv = _dinv(degp_ref)
    s2 = p_ref[0] + p_ref[1] + g2_ref[...]
    a2 = s2 * dinv[:, None] + b2_ref[...][None, :]
    m = jnp.max(a2, axis=1, keepdims=True)
    e = jnp.exp(a2 - m)
    lse = jnp.log(jnp.sum(e, axis=1, keepdims=True))
    out_ref[...] = a2 - m - lse


_tc1 = pl.pallas_call(
    _tc1_body, out_shape=jax.ShapeDtypeStruct((N_PAD, D), jnp.float32))
_tc2 = pl.pallas_call(
    _tc2_body, out_shape=jax.ShapeDtypeStruct((N_PAD, D), jnp.float32))
_tc3 = pl.pallas_call(
    _tc3_body, out_shape=jax.ShapeDtypeStruct((N_PAD, D), jnp.float32))


def kernel(x, edge_index, W1, b1, W2, b2):
    n = x.shape[0]
    x_pad = jnp.zeros((N_PAD, D_IN), jnp.float32).at[:n].set(x)

    n_extra = E_PAD - edge_index.shape[1]
    pad_ids = n + (jnp.arange(n_extra, dtype=jnp.int32) % (N_PAD - n))
    src2d = jnp.concatenate([edge_index[0], pad_ids]).reshape(E_ROWS, EW)
    dst2d = jnp.concatenate([edge_index[1], pad_ids]).reshape(E_ROWS, EW)

    zeros1 = jnp.zeros((N_PAD,), jnp.float32)
    zeros2 = jnp.zeros((N_PAD, D), jnp.float32)

    degp = _sc_degree(dst2d, zeros1)
    g1 = _tc1(x_pad, W1, degp)
    p1 = _sc_aggregate(g1, src2d, dst2d, zeros2)
    g2 = _tc2(p1, g1, degp, b1, W2)
    p2 = _sc_aggregate(g2, src2d, dst2d, zeros2)
    out = _tc3(p2, g2, degp, b2)
    return out[:n]


# speedup vs baseline: 50.4100x; 1.3571x over previous
import functools

import jax
import jax.numpy as jnp
from jax import lax
from jax.experimental import pallas as pl
from jax.experimental.pallas import tpu as pltpu
from jax.experimental.pallas import tpu_sc as plsc

N_NODES = 10000
D_IN = 128
D = 16

NC = 2
NS = 16
NW = NC * NS

N_PAD = 10240
CHUNK = N_PAD // NS

EW = 128
E_EDGES = 320000
ROWS_PER_TILE = 80
PER_TILE = ROWS_PER_TILE * EW
E_PAD = NW * PER_TILE
E_ROWS = E_PAD // EW

_mesh = plsc.VectorSubcoreMesh(core_axis_name="c", subcore_axis_name="s")


@functools.partial(
    pl.kernel,
    out_type=jax.ShapeDtypeStruct((NC, N_PAD), jnp.float32),
    mesh=_mesh,
    scratch_types=[
        pltpu.VMEM((ROWS_PER_TILE, EW), jnp.int32),
        pltpu.VMEM((EW,), jnp.float32),
        pltpu.VMEM_SHARED((N_PAD,), jnp.float32),
    ],
)
def _sc_degree(dst_hbm, zeros1_hbm, out_hbm, didx_v, ones_v, deg_sh):
    c = lax.axis_index("c")
    s = lax.axis_index("s")
    w = c * NS + s
    pltpu.sync_copy(zeros1_hbm.at[pl.ds(s * CHUNK, CHUNK)],
                    deg_sh.at[pl.ds(s * CHUNK, CHUNK)])
    pltpu.sync_copy(dst_hbm.at[pl.ds(w * ROWS_PER_TILE, ROWS_PER_TILE)], didx_v)
    for i in range(EW // 16):
        ones_v[pl.ds(i * 16, 16)] = jnp.ones((16,), jnp.float32)
    plsc.subcore_barrier()

    def body(j, carry):
        pltpu.sync_copy(ones_v, deg_sh.at[didx_v.at[j]], add=True)
        return carry

    lax.fori_loop(0, ROWS_PER_TILE, body, 0)
    plsc.subcore_barrier()
    pltpu.sync_copy(deg_sh.at[pl.ds(s * CHUNK, CHUNK)],
                    out_hbm.at[c, pl.ds(s * CHUNK, CHUNK)])


@functools.partial(
    pl.kernel,
    out_type=jax.ShapeDtypeStruct((NC, N_PAD, D), jnp.float32),
    mesh=_mesh,
    scratch_types=[
        pltpu.VMEM((ROWS_PER_TILE, EW), jnp.int32),
        pltpu.VMEM((ROWS_PER_TILE, EW), jnp.int32),
        pltpu.VMEM((EW, D), jnp.float32),
        pltpu.VMEM((EW, D), jnp.float32),
        pltpu.VMEM_SHARED((N_PAD, D), jnp.float32),
        pltpu.SemaphoreType.DMA,
        pltpu.SemaphoreType.DMA,
    ],
    compiler_params=pltpu.CompilerParams(use_tc_tiling_on_sc=False),
)
def _sc_aggregate(g_hbm, src_hbm, dst_hbm, zeros2_hbm, out_hbm,
                  sidx_v, didx_v, rows0_v, rows1_v, acc_sh, sem0, sem1):
    c = lax.axis_index("c")
    s = lax.axis_index("s")
    w = c * NS + s
    pltpu.sync_copy(zeros2_hbm.at[pl.ds(s * CHUNK, CHUNK)],
                    acc_sh.at[pl.ds(s * CHUNK, CHUNK)])
    pltpu.sync_copy(src_hbm.at[pl.ds(w * ROWS_PER_TILE, ROWS_PER_TILE)], sidx_v)
    pltpu.sync_copy(dst_hbm.at[pl.ds(w * ROWS_PER_TILE, ROWS_PER_TILE)], didx_v)
    plsc.subcore_barrier()

    pltpu.async_copy(g_hbm.at[sidx_v.at[0]], rows0_v, sem0)

    def body(i, carry):
        j = 2 * i
        pltpu.async_copy(g_hbm.at[sidx_v.at[j + 1]], rows1_v, sem1)
        pltpu.make_async_copy(g_hbm.at[sidx_v.at[j]], rows0_v, sem0).wait()
        pltpu.sync_copy(rows0_v, acc_sh.at[didx_v.at[j]], add=True)

        @pl.when(j + 2 < ROWS_PER_TILE)
        def _():
            pltpu.async_copy(g_hbm.at[sidx_v.at[j + 2]], rows0_v, sem0)

        pltpu.make_async_copy(g_hbm.at[sidx_v.at[j + 1]], rows1_v, sem1).wait()
        pltpu.sync_copy(rows1_v, acc_sh.at[didx_v.at[j + 1]], add=True)
        return carry

    lax.fori_loop(0, ROWS_PER_TILE // 2, body, 0)
    plsc.subcore_barrier()
    pltpu.sync_copy(acc_sh.at[pl.ds(s * CHUNK, CHUNK)],
                    out_hbm.at[c, pl.ds(s * CHUNK, CHUNK)])


def _dinv(degp_ref):
    deg = degp_ref[0, :] + degp_ref[1, :] + 1.0
    return lax.rsqrt(deg)


def _tc1_body(x_ref, w1_ref, degp_ref, g1_ref):
    dinv = _dinv(degp_ref)
    h = jnp.dot(x_ref[...], w1_ref[...], preferred_element_type=jnp.float32)
    g1_ref[...] = h * dinv[:, None]


def _tc2_body(p_ref, g1_ref, degp_ref, b1_ref, w2_ref, g2_ref):
    dinv = _dinv(degp_ref)
    s1 = p_ref[0] + p_ref[1] + g1_ref[...]
    a1 = s1 * dinv[:, None] + b1_ref[...][None, :]
    h1 = jnp.maximum(a1, 0.0)
    h2 = jnp.dot(h1, w2_ref[...], preferred_element_type=jnp.float32)
    g2_ref[...] = h2 * dinv[:, None]


def _tc3_body(p_ref, g2_ref, degp_ref, b2_ref, out_ref):
    dinv = _dinv(degp_ref)
    s2 = p_ref[0] + p_ref[1] + g2_ref[...]
    a2 = s2 * dinv[:, None] + b2_ref[...][None, :]
    m = jnp.max(a2, axis=1, keepdims=True)
    e = jnp.exp(a2 - m)
    lse = jnp.log(jnp.sum(e, axis=1, keepdims=True))
    out_ref[...] = a2 - m - lse


_tc1 = pl.pallas_call(
    _tc1_body, out_shape=jax.ShapeDtypeStruct((N_PAD, D), jnp.float32))
_tc2 = pl.pallas_call(
    _tc2_body, out_shape=jax.ShapeDtypeStruct((N_PAD, D), jnp.float32))
_tc3 = pl.pallas_call(
    _tc3_body, out_shape=jax.ShapeDtypeStruct((N_PAD, D), jnp.float32))


def kernel(x, edge_index, W1, b1, W2, b2):
    n = x.shape[0]
    x_pad = jnp.zeros((N_PAD, D_IN), jnp.float32).at[:n].set(x)

    n_extra = E_PAD - edge_index.shape[1]
    pad_ids = n + (jnp.arange(n_extra, dtype=jnp.int32) % (N_PAD - n))
    src2d = jnp.concatenate([edge_index[0], pad_ids]).reshape(E_ROWS, EW)
    dst2d = jnp.concatenate([edge_index[1], pad_ids]).reshape(E_ROWS, EW)

    zeros1 = jnp.zeros((N_PAD,), jnp.float32)
    zeros2 = jnp.zeros((N_PAD, D), jnp.float32)

    degp = _sc_degree(dst2d, zeros1)
    g1 = _tc1(x_pad, W1, degp)
    p1 = _sc_aggregate(g1, src2d, dst2d, zeros2)
    g2 = _tc2(p1, g1, degp, b1, W2)
    p2 = _sc_aggregate(g2, src2d, dst2d, zeros2)
    out = _tc3(p2, g2, degp, b2)
    return out[:n]


# speedup vs baseline: 64.5606x; 1.2807x over previous
import functools

import jax
import jax.numpy as jnp
from jax import lax
from jax.experimental import pallas as pl
from jax.experimental.pallas import tpu as pltpu
from jax.experimental.pallas import tpu_sc as plsc

N_NODES = 10000
D_IN = 128
D = 16

NC = 2
NS = 16
NW = NC * NS

N_PAD = 10240
CHUNK = N_PAD // NS

EW = 128
E_EDGES = 320000
ROWS_PER_TILE = 80
PER_TILE = ROWS_PER_TILE * EW
E_PAD = NW * PER_TILE
E_ROWS = E_PAD // EW

_mesh = plsc.VectorSubcoreMesh(core_axis_name="c", subcore_axis_name="s")


@functools.partial(
    pl.kernel,
    out_type=jax.ShapeDtypeStruct((NC, N_PAD), jnp.float32),
    mesh=_mesh,
    scratch_types=[
        pltpu.VMEM((ROWS_PER_TILE, EW), jnp.int32),
        pltpu.VMEM((EW,), jnp.float32),
        pltpu.VMEM_SHARED((N_PAD,), jnp.float32),
        pltpu.SemaphoreType.DMA,
    ],
)
def _sc_degree(dst_hbm, zeros1_hbm, out_hbm, didx_v, ones_v, deg_sh, sem):
    c = lax.axis_index("c")
    s = lax.axis_index("s")
    w = c * NS + s
    pltpu.sync_copy(zeros1_hbm.at[pl.ds(s * CHUNK, CHUNK)],
                    deg_sh.at[pl.ds(s * CHUNK, CHUNK)])
    pltpu.sync_copy(dst_hbm.at[pl.ds(w * ROWS_PER_TILE, ROWS_PER_TILE)], didx_v)
    for i in range(EW // 16):
        ones_v[pl.ds(i * 16, 16)] = jnp.ones((16,), jnp.float32)
    plsc.subcore_barrier()

    def body(j, carry):
        pltpu.async_copy(ones_v, deg_sh.at[didx_v.at[j]], sem, add=True)
        return carry

    lax.fori_loop(0, ROWS_PER_TILE, body, 0)

    def drain(j, carry):
        pltpu.make_async_copy(ones_v, deg_sh.at[didx_v.at[0]], sem).wait()
        return carry

    lax.fori_loop(0, ROWS_PER_TILE, drain, 0)
    plsc.subcore_barrier()
    pltpu.sync_copy(deg_sh.at[pl.ds(s * CHUNK, CHUNK)],
                    out_hbm.at[c, pl.ds(s * CHUNK, CHUNK)])


@functools.partial(
    pl.kernel,
    out_type=jax.ShapeDtypeStruct((NC, N_PAD, D), jnp.float32),
    mesh=_mesh,
    scratch_types=[
        pltpu.VMEM((ROWS_PER_TILE, EW), jnp.int32),
        pltpu.VMEM((ROWS_PER_TILE, EW), jnp.int32),
        [pltpu.VMEM((EW, D), jnp.float32)] * 8,
        [pltpu.SemaphoreType.DMA] * 8,
        [pltpu.SemaphoreType.DMA] * 8,
        pltpu.VMEM_SHARED((N_PAD, D), jnp.float32),
    ],
    compiler_params=pltpu.CompilerParams(use_tc_tiling_on_sc=False),
)
def _sc_aggregate(g_hbm, src_hbm, dst_hbm, zeros2_hbm, out_hbm,
                  sidx_v, didx_v, rows, gsem, ssem, acc_sh):
    c = lax.axis_index("c")
    s = lax.axis_index("s")
    w = c * NS + s
    pltpu.sync_copy(zeros2_hbm.at[pl.ds(s * CHUNK, CHUNK)],
                    acc_sh.at[pl.ds(s * CHUNK, CHUNK)])
    pltpu.sync_copy(src_hbm.at[pl.ds(w * ROWS_PER_TILE, ROWS_PER_TILE)], sidx_v)
    pltpu.sync_copy(dst_hbm.at[pl.ds(w * ROWS_PER_TILE, ROWS_PER_TILE)], didx_v)
    plsc.subcore_barrier()

    for b in range(4):
        pltpu.async_copy(g_hbm.at[sidx_v.at[b]], rows[b], gsem[b])

    def body(i, carry):
        for b in range(8):
            j = 8 * i + b
            bb = (b + 4) % 8
            pltpu.make_async_copy(g_hbm.at[sidx_v.at[j]], rows[b],
                                  gsem[b]).wait()
            pltpu.async_copy(rows[b], acc_sh.at[didx_v.at[j]], ssem[b],
                             add=True)

            @pl.when(j + 4 < ROWS_PER_TILE)
            def _():
                @pl.when(j + 4 >= 8)
                def _():
                    pltpu.make_async_copy(rows[bb], acc_sh.at[didx_v.at[0]],
                                          ssem[bb]).wait()

                pltpu.async_copy(g_hbm.at[sidx_v.at[j + 4]], rows[bb],
                                 gsem[bb])

        return carry

    lax.fori_loop(0, ROWS_PER_TILE // 8, body, 0)
    for b in range(8):
        pltpu.make_async_copy(rows[b], acc_sh.at[didx_v.at[0]],
                              ssem[b]).wait()
    plsc.subcore_barrier()
    pltpu.sync_copy(acc_sh.at[pl.ds(s * CHUNK, CHUNK)],
                    out_hbm.at[c, pl.ds(s * CHUNK, CHUNK)])


def _dinv(degp_ref):
    deg = degp_ref[0, :] + degp_ref[1, :] + 1.0
    return lax.rsqrt(deg)


def _tc1_body(x_ref, w1_ref, degp_ref, g1_ref):
    dinv = _dinv(degp_ref)
    h = jnp.dot(x_ref[...], w1_ref[...], preferred_element_type=jnp.float32)
    g1_ref[...] = h * dinv[:, None]


def _tc2_body(p_ref, g1_ref, degp_ref, b1_ref, w2_ref, g2_ref):
    dinv = _dinv(degp_ref)
    s1 = p_ref[0] + p_ref[1] + g1_ref[...]
    a1 = s1 * dinv[:, None] + b1_ref[...][None, :]
    h1 = jnp.maximum(a1, 0.0)
    h2 = jnp.dot(h1, w2_ref[...], preferred_element_type=jnp.float32)
    g2_ref[...] = h2 * dinv[:, None]


def _tc3_body(p_ref, g2_ref, degp_ref, b2_ref, out_ref):
    dinv = _dinv(degp_ref)
    s2 = p_ref[0] + p_ref[1] + g2_ref[...]
    a2 = s2 * dinv[:, None] + b2_ref[...][None, :]
    m = jnp.max(a2, axis=1, keepdims=True)
    e = jnp.exp(a2 - m)
    lse = jnp.log(jnp.sum(e, axis=1, keepdims=True))
    out_ref[...] = a2 - m - lse


_tc1 = pl.pallas_call(
    _tc1_body, out_shape=jax.ShapeDtypeStruct((N_PAD, D), jnp.float32))
_tc2 = pl.pallas_call(
    _tc2_body, out_shape=jax.ShapeDtypeStruct((N_PAD, D), jnp.float32))
_tc3 = pl.pallas_call(
    _tc3_body, out_shape=jax.ShapeDtypeStruct((N_PAD, D), jnp.float32))


def kernel(x, edge_index, W1, b1, W2, b2):
    n = x.shape[0]
    x_pad = jnp.zeros((N_PAD, D_IN), jnp.float32).at[:n].set(x)

    n_extra = E_PAD - edge_index.shape[1]
    pad_ids = n + (jnp.arange(n_extra, dtype=jnp.int32) % (N_PAD - n))
    src2d = jnp.concatenate([edge_index[0], pad_ids]).reshape(E_ROWS, EW)
    dst2d = jnp.concatenate([edge_index[1], pad_ids]).reshape(E_ROWS, EW)

    zeros1 = jnp.zeros((N_PAD,), jnp.float32)
    zeros2 = jnp.zeros((N_PAD, D), jnp.float32)

    degp = _sc_degree(dst2d, zeros1)
    g1 = _tc1(x_pad, W1, degp)
    p1 = _sc_aggregate(g1, src2d, dst2d, zeros2)
    g2 = _tc2(p1, g1, degp, b1, W2)
    p2 = _sc_aggregate(g2, src2d, dst2d, zeros2)
    out = _tc3(p2, g2, degp, b2)
    return out[:n]


# speedup vs baseline: 69.6875x; 1.0794x over previous
import functools

import jax
import jax.numpy as jnp
from jax import lax
from jax.experimental import pallas as pl
from jax.experimental.pallas import tpu as pltpu
from jax.experimental.pallas import tpu_sc as plsc

N_NODES = 10000
D_IN = 128
D = 16

NC = 2
NS = 16
NW = NC * NS

N_PAD = 10240
CHUNK = N_PAD // NS

EW = 128
E_EDGES = 320000
W_TOT = E_EDGES // EW
W_BASE = W_TOT // NW
W_EXTRA = W_TOT - W_BASE * NW
W_MAX = W_BASE + 1

_mesh = plsc.VectorSubcoreMesh(core_axis_name="c", subcore_axis_name="s")
_sc_params = pltpu.CompilerParams(use_tc_tiling_on_sc=False)


def _worker_windows(w):
    cnt = jnp.where(w < W_EXTRA, W_BASE + 1, W_BASE)
    base = W_BASE * w + jnp.minimum(w, W_EXTRA)
    clamped = jnp.minimum(base, W_TOT - W_MAX)
    return clamped, base - clamped, cnt


@functools.partial(
    pl.kernel,
    out_type=jax.ShapeDtypeStruct((NC, N_PAD), jnp.float32),
    mesh=_mesh,
    scratch_types=[
        pltpu.VMEM((W_MAX, EW), jnp.int32),
        pltpu.VMEM((EW,), jnp.float32),
        pltpu.VMEM_SHARED((N_PAD,), jnp.float32),
        pltpu.SemaphoreType.DMA,
    ],
    compiler_params=_sc_params,
)
def _sc_degree(ei_hbm, zeros1_hbm, out_hbm, didx_v, ones_v, deg_sh, sem):
    c = lax.axis_index("c")
    s = lax.axis_index("s")
    w = c * NS + s
    clamped, shift, cnt = _worker_windows(w)
    pltpu.sync_copy(zeros1_hbm.at[pl.ds(s * CHUNK, CHUNK)],
                    deg_sh.at[pl.ds(s * CHUNK, CHUNK)])
    pltpu.sync_copy(ei_hbm.at[1, pl.ds(clamped, W_MAX)], didx_v)
    for i in range(EW // 16):
        ones_v[pl.ds(i * 16, 16)] = jnp.ones((16,), jnp.float32)
    plsc.subcore_barrier()

    def body(j, carry):
        pltpu.async_copy(ones_v, deg_sh.at[didx_v.at[shift + j]], sem,
                         add=True)
        return carry

    lax.fori_loop(0, cnt, body, 0)

    def drain(j, carry):
        pltpu.make_async_copy(ones_v, deg_sh.at[didx_v.at[0]], sem).wait()
        return carry

    lax.fori_loop(0, cnt, drain, 0)
    plsc.subcore_barrier()
    pltpu.sync_copy(deg_sh.at[pl.ds(s * CHUNK, CHUNK)],
                    out_hbm.at[c, pl.ds(s * CHUNK, CHUNK)])


@functools.partial(
    pl.kernel,
    out_type=jax.ShapeDtypeStruct((NC, N_PAD, D), jnp.float32),
    mesh=_mesh,
    scratch_types=[
        pltpu.VMEM((W_MAX, EW), jnp.int32),
        pltpu.VMEM((W_MAX, EW), jnp.int32),
        [pltpu.VMEM((EW, D), jnp.float32)] * 8,
        [pltpu.SemaphoreType.DMA] * 8,
        [pltpu.SemaphoreType.DMA] * 8,
        pltpu.VMEM_SHARED((N_PAD, D), jnp.float32),
    ],
    compiler_params=_sc_params,
)
def _sc_aggregate(g_hbm, ei_hbm, zeros2_hbm, out_hbm,
                  sidx_v, didx_v, rows, gsem, ssem, acc_sh):
    c = lax.axis_index("c")
    s = lax.axis_index("s")
    w = c * NS + s
    clamped, shift, cnt = _worker_windows(w)
    pltpu.sync_copy(zeros2_hbm.at[pl.ds(s * CHUNK, CHUNK)],
                    acc_sh.at[pl.ds(s * CHUNK, CHUNK)])
    pltpu.sync_copy(ei_hbm.at[0, pl.ds(clamped, W_MAX)], sidx_v)
    pltpu.sync_copy(ei_hbm.at[1, pl.ds(clamped, W_MAX)], didx_v)
    plsc.subcore_barrier()

    for b in range(4):
        pltpu.async_copy(g_hbm.at[sidx_v.at[shift + b]], rows[b], gsem[b])

    def body(i, carry):
        for b in range(8):
            j = 8 * i + b
            bb = (b + 4) % 8

            @pl.when(j < cnt)
            def _():
                pltpu.make_async_copy(g_hbm.at[sidx_v.at[shift + j]], rows[b],
                                      gsem[b]).wait()
                pltpu.async_copy(rows[b], acc_sh.at[didx_v.at[shift + j]],
                                 ssem[b], add=True)

            @pl.when(j + 4 < cnt)
            def _():
                @pl.when(j + 4 >= 8)
                def _():
                    pltpu.make_async_copy(rows[bb], acc_sh.at[didx_v.at[0]],
                                          ssem[bb]).wait()

                pltpu.async_copy(g_hbm.at[sidx_v.at[shift + j + 4]], rows[bb],
                                 gsem[bb])

        return carry

    lax.fori_loop(0, (W_MAX + 7) // 8, body, 0)
    for b in range(8):
        pltpu.make_async_copy(rows[b], acc_sh.at[didx_v.at[0]],
                              ssem[b]).wait()
    plsc.subcore_barrier()
    pltpu.sync_copy(acc_sh.at[pl.ds(s * CHUNK, CHUNK)],
                    out_hbm.at[c, pl.ds(s * CHUNK, CHUNK)])


def _dinv(degp_ref):
    deg = degp_ref[0, :] + degp_ref[1, :] + 1.0
    return lax.rsqrt(deg)


def _tc1_body(x_ref, w1_ref, degp_ref, g1_ref):
    dinv = _dinv(degp_ref)
    h = jnp.dot(x_ref[...], w1_ref[...], preferred_element_type=jnp.float32)
    g1_ref[:N_NODES, :] = h * dinv[:N_NODES, None]
    g1_ref[N_NODES:, :] = jnp.zeros((N_PAD - N_NODES, D), jnp.float32)


def _tc2_body(p_ref, g1_ref, degp_ref, b1_ref, w2_ref, g2_ref):
    dinv = _dinv(degp_ref)
    s1 = p_ref[0] + p_ref[1] + g1_ref[...]
    a1 = s1 * dinv[:, None] + b1_ref[...][None, :]
    h1 = jnp.maximum(a1, 0.0)
    h2 = jnp.dot(h1, w2_ref[...], preferred_element_type=jnp.float32)
    g2_ref[...] = h2 * dinv[:, None]


def _tc3_body(p_ref, g2_ref, degp_ref, b2_ref, out_ref):
    dinv = _dinv(degp_ref)
    s2 = p_ref[0] + p_ref[1] + g2_ref[...]
    a2 = s2 * dinv[:, None] + b2_ref[...][None, :]
    m = jnp.max(a2, axis=1, keepdims=True)
    e = jnp.exp(a2 - m)
    lse = jnp.log(jnp.sum(e, axis=1, keepdims=True))
    out_ref[...] = a2 - m - lse


_tc1 = pl.pallas_call(
    _tc1_body, out_shape=jax.ShapeDtypeStruct((N_PAD, D), jnp.float32))
_tc2 = pl.pallas_call(
    _tc2_body, out_shape=jax.ShapeDtypeStruct((N_PAD, D), jnp.float32))
_tc3 = pl.pallas_call(
    _tc3_body, out_shape=jax.ShapeDtypeStruct((N_PAD, D), jnp.float32))


def kernel(x, edge_index, W1, b1, W2, b2):
    n = x.shape[0]
    ei3 = edge_index.reshape(2, W_TOT, EW)

    zeros1 = jnp.zeros((N_PAD,), jnp.float32)
    zeros2 = jnp.zeros((N_PAD, D), jnp.float32)

    degp = _sc_degree(ei3, zeros1)
    g1 = _tc1(x, W1, degp)
    p1 = _sc_aggregate(g1, ei3, zeros2)
    g2 = _tc2(p1, g1, degp, b1, W2)
    p2 = _sc_aggregate(g2, ei3, zeros2)
    out = _tc3(p2, g2, degp, b2)
    return out[:n]


# speedup vs baseline: 98.8414x; 1.4184x over previous
import functools

import jax
import jax.numpy as jnp
from jax import lax
from jax.experimental import pallas as pl
from jax.experimental.pallas import tpu as pltpu
from jax.experimental.pallas import tpu_sc as plsc

N_NODES = 10000
D_IN = 128
D = 16

NC = 2
NS = 16
NW = NC * NS

N_PAD = 10240
CHUNK = N_PAD // NS
F_ROWS = N_PAD * D // 128
F_REAL = N_NODES * D // 128

EW = 128
E_EDGES = 320000
W_TOT = E_EDGES // EW
W_BASE = W_TOT // NW
W_EXTRA = W_TOT - W_BASE * NW
W_MAX = W_BASE + 1

_mesh = plsc.VectorSubcoreMesh(core_axis_name="c", subcore_axis_name="s")
_sc_params = pltpu.CompilerParams(use_tc_tiling_on_sc=False)


def _worker_windows(w):
    cnt = jnp.where(w < W_EXTRA, W_BASE + 1, W_BASE)
    base = W_BASE * w + jnp.minimum(w, W_EXTRA)
    clamped = jnp.minimum(base, W_TOT - W_MAX)
    return clamped, base - clamped, cnt


@functools.partial(
    pl.kernel,
    out_type=jax.ShapeDtypeStruct((NC, N_PAD), jnp.float32),
    mesh=_mesh,
    scratch_types=[
        pltpu.VMEM((W_MAX, 1, EW), jnp.int32),
        pltpu.VMEM((EW,), jnp.float32),
        pltpu.VMEM_SHARED((N_PAD,), jnp.float32),
        pltpu.SemaphoreType.DMA,
    ],
    compiler_params=_sc_params,
)
def _sc_degree(ei_hbm, zeros1_hbm, out_hbm, didx_v, ones_v, deg_sh, sem):
    c = lax.axis_index("c")
    s = lax.axis_index("s")
    w = c * NS + s
    clamped, shift, cnt = _worker_windows(w)
    pltpu.sync_copy(zeros1_hbm.at[pl.ds(s * CHUNK, CHUNK)],
                    deg_sh.at[pl.ds(s * CHUNK, CHUNK)])
    pltpu.sync_copy(ei_hbm.at[pl.ds(clamped, W_MAX), pl.ds(1, 1)], didx_v)
    for i in range(EW // 16):
        ones_v[pl.ds(i * 16, 16)] = jnp.ones((16,), jnp.float32)
    plsc.subcore_barrier()

    def body(j, carry):
        pltpu.async_copy(ones_v, deg_sh.at[didx_v.at[shift + j, 0]], sem,
                         add=True)
        return carry

    lax.fori_loop(0, cnt, body, 0)

    def drain(j, carry):
        pltpu.make_async_copy(ones_v, deg_sh.at[didx_v.at[0, 0]], sem).wait()
        return carry

    lax.fori_loop(0, cnt, drain, 0)
    plsc.subcore_barrier()
    pltpu.sync_copy(deg_sh.at[pl.ds(s * CHUNK, CHUNK)],
                    out_hbm.at[c, pl.ds(s * CHUNK, CHUNK)])


@functools.partial(
    pl.kernel,
    out_type=jax.ShapeDtypeStruct((NC, N_PAD, D), jnp.float32),
    mesh=_mesh,
    scratch_types=[
        pltpu.VMEM((W_MAX, 1, EW), jnp.int32),
        pltpu.VMEM((W_MAX, 1, EW), jnp.int32),
        [pltpu.VMEM((EW, D), jnp.float32)] * 8,
        [pltpu.SemaphoreType.DMA] * 8,
        [pltpu.SemaphoreType.DMA] * 8,
        pltpu.VMEM_SHARED((N_PAD, D), jnp.float32),
    ],
    compiler_params=_sc_params,
)
def _sc_aggregate(g_hbm, ei_hbm, zeros2_hbm, out_hbm,
                  sidx_v, didx_v, rows, gsem, ssem, acc_sh):
    c = lax.axis_index("c")
    s = lax.axis_index("s")
    w = c * NS + s
    clamped, shift, cnt = _worker_windows(w)
    pltpu.sync_copy(zeros2_hbm.at[pl.ds(s * CHUNK, CHUNK)],
                    acc_sh.at[pl.ds(s * CHUNK, CHUNK)])
    pltpu.sync_copy(ei_hbm.at[pl.ds(clamped, W_MAX), pl.ds(0, 1)], sidx_v)
    pltpu.sync_copy(ei_hbm.at[pl.ds(clamped, W_MAX), pl.ds(1, 1)], didx_v)
    plsc.subcore_barrier()

    for b in range(4):
        pltpu.async_copy(g_hbm.at[sidx_v.at[shift + b, 0]], rows[b], gsem[b])

    def body(i, carry):
        for b in range(8):
            j = 8 * i + b
            bb = (b + 4) % 8

            @pl.when(j < cnt)
            def _():
                pltpu.make_async_copy(g_hbm.at[sidx_v.at[shift + j, 0]],
                                      rows[b], gsem[b]).wait()
                pltpu.async_copy(rows[b], acc_sh.at[didx_v.at[shift + j, 0]],
                                 ssem[b], add=True)

            @pl.when(j + 4 < cnt)
            def _():
                @pl.when(j + 4 >= 8)
                def _():
                    pltpu.make_async_copy(rows[bb],
                                          acc_sh.at[didx_v.at[0, 0]],
                                          ssem[bb]).wait()

                pltpu.async_copy(g_hbm.at[sidx_v.at[shift + j + 4, 0]],
                                 rows[bb], gsem[bb])

        return carry

    lax.fori_loop(0, (W_MAX + 7) // 8, body, 0)
    for b in range(8):
        pltpu.make_async_copy(rows[b], acc_sh.at[didx_v.at[0, 0]],
                              ssem[b]).wait()
    plsc.subcore_barrier()
    pltpu.sync_copy(acc_sh.at[pl.ds(s * CHUNK, CHUNK)],
                    out_hbm.at[c, pl.ds(s * CHUNK, CHUNK)])


def _tc1_body(x_ref, w1s_ref, mt_ref, degp_ref, g1f_ref, dinvf_ref):
    deg80 = degp_ref[0] + degp_ref[1] + 1.0
    dinv80 = lax.rsqrt(deg80)
    parts = [jnp.dot(dinv80, mt_ref[t], preferred_element_type=jnp.float32)
             for t in range(16)]
    dinvf = jnp.stack(parts, axis=1).reshape(F_ROWS, 128)
    dinvf_ref[...] = dinvf
    x3 = x_ref[...].reshape(F_REAL, 8, D_IN)
    hf = jnp.zeros((F_REAL, 128), jnp.float32)
    for k in range(8):
        hf = hf + jnp.dot(x3[:, k, :], w1s_ref[k],
                          preferred_element_type=jnp.float32)
    g1f_ref[:F_REAL, :] = hf * dinvf[:F_REAL, :]
    g1f_ref[F_REAL:, :] = jnp.zeros((F_ROWS - F_REAL, 128), jnp.float32)


def _tc2_body(p_ref, g1f_ref, dinvf_ref, b1f_ref, w2b_ref, g2f_ref):
    dinvf = dinvf_ref[...]
    s1 = p_ref[0] + p_ref[1] + g1f_ref[...]
    a1 = s1 * dinvf + b1f_ref[...][None, :]
    h1 = jnp.maximum(a1, 0.0)
    h2 = jnp.dot(h1, w2b_ref[...], preferred_element_type=jnp.float32)
    g2f_ref[...] = h2 * dinvf


def _tc3_body(p_ref, g2f_ref, dinvf_ref, b2f_ref, mavg_ref, msum_ref,
              out_ref):
    dinvf = dinvf_ref[...]
    s2 = p_ref[0] + p_ref[1] + g2f_ref[...]
    a2f = s2 * dinvf + b2f_ref[...][None, :]
    mf = jnp.dot(a2f, mavg_ref[...], preferred_element_type=jnp.float32)
    ef = jnp.exp(a2f - mf)
    sf = jnp.dot(ef, msum_ref[...], preferred_element_type=jnp.float32)
    out_ref[...] = a2f - mf - jnp.log(sf)


_tc1 = pl.pallas_call(
    _tc1_body,
    out_shape=(jax.ShapeDtypeStruct((F_ROWS, 128), jnp.float32),
               jax.ShapeDtypeStruct((F_ROWS, 128), jnp.float32)))
_tc2 = pl.pallas_call(
    _tc2_body, out_shape=jax.ShapeDtypeStruct((F_ROWS, 128), jnp.float32))
_tc3 = pl.pallas_call(
    _tc3_body, out_shape=jax.ShapeDtypeStruct((F_ROWS, 128), jnp.float32))


def kernel(x, edge_index, W1, b1, W2, b2):
    ei_t = edge_index.reshape(2, W_TOT, EW).transpose(1, 0, 2)

    lanes = jnp.arange(128, dtype=jnp.int32)
    kmask = (lanes[None, None, :] // D == jnp.arange(8, dtype=jnp.int32)[:, None, None])
    W1S = jnp.tile(W1, (1, 8))[None, :, :] * kmask.astype(jnp.float32)
    W2b = jnp.kron(jnp.eye(8, dtype=W2.dtype), W2)
    Mt = (jnp.arange(128, dtype=jnp.int32)[None, :, None]
          == 8 * jnp.arange(16, dtype=jnp.int32)[:, None, None]
          + lanes[None, None, :] // D).astype(jnp.float32)
    seg = (lanes[:, None] // D == lanes[None, :] // D).astype(jnp.float32)
    Mavg = seg / float(D)
    Msum = seg
    b1f = jnp.tile(b1, 8)
    b2f = jnp.tile(b2, 8)

    zeros1 = jnp.zeros((N_PAD,), jnp.float32)
    zeros2 = jnp.zeros((N_PAD, D), jnp.float32)

    degp = _sc_degree(ei_t, zeros1)
    g1f, dinvf = _tc1(x, W1S, Mt, degp.reshape(2, 80, 128))
    p1 = _sc_aggregate(g1f.reshape(N_PAD, D), ei_t, zeros2)
    g2f = _tc2(p1.reshape(NC, F_ROWS, 128), g1f, dinvf, b1f, W2b)
    p2 = _sc_aggregate(g2f.reshape(N_PAD, D), ei_t, zeros2)
    outf = _tc3(p2.reshape(NC, F_ROWS, 128), g2f, dinvf, b2f, Mavg, Msum)
    return outf.reshape(N_PAD, D)[:N_NODES]


# speedup vs baseline: 102.7134x; 1.0392x over previous
import functools

import jax
import jax.numpy as jnp
from jax import lax
from jax.experimental import pallas as pl
from jax.experimental.pallas import tpu as pltpu
from jax.experimental.pallas import tpu_sc as plsc

N_NODES = 10000
D_IN = 128
D = 16

NC = 2
NS = 16
NW = NC * NS

N_PAD = 10240
CHUNK = N_PAD // NS
F_ROWS = N_PAD * D // 128
F_REAL = N_NODES * D // 128

EW = 128
E_EDGES = 320000
W_TOT = E_EDGES // EW
W_BASE = W_TOT // NW
W_EXTRA = W_TOT - W_BASE * NW
W_MAX = W_BASE + 1

_mesh = plsc.VectorSubcoreMesh(core_axis_name="c", subcore_axis_name="s")
_sc_params = pltpu.CompilerParams(use_tc_tiling_on_sc=False)


def _worker_windows(w):
    cnt = jnp.where(w < W_EXTRA, W_BASE + 1, W_BASE)
    base = W_BASE * w + jnp.minimum(w, W_EXTRA)
    clamped = jnp.minimum(base, W_TOT - W_MAX)
    return clamped, base - clamped, cnt


@functools.partial(
    pl.kernel,
    out_type=jax.ShapeDtypeStruct((NC, N_PAD), jnp.float32),
    mesh=_mesh,
    scratch_types=[
        pltpu.VMEM((W_MAX, 1, EW), jnp.int32),
        pltpu.VMEM((EW,), jnp.float32),
        pltpu.VMEM_SHARED((N_PAD,), jnp.float32),
        pltpu.SemaphoreType.DMA,
    ],
    compiler_params=_sc_params,
)
def _sc_degree(ei_hbm, zeros1_hbm, out_hbm, didx_v, ones_v, deg_sh, sem):
    c = lax.axis_index("c")
    s = lax.axis_index("s")
    w = c * NS + s
    clamped, shift, cnt = _worker_windows(w)
    d0 = pltpu.async_copy(zeros1_hbm.at[pl.ds(s * CHUNK, CHUNK)],
                          deg_sh.at[pl.ds(s * CHUNK, CHUNK)], sem)
    d1 = pltpu.async_copy(ei_hbm.at[pl.ds(clamped, W_MAX), pl.ds(1, 1)],
                          didx_v, sem)
    d0.wait()
    d1.wait()
    for i in range(EW // 16):
        ones_v[pl.ds(i * 16, 16)] = jnp.ones((16,), jnp.float32)
    plsc.subcore_barrier()

    def body(j, carry):
        pltpu.async_copy(ones_v, deg_sh.at[didx_v.at[shift + j, 0]], sem,
                         add=True)
        return carry

    lax.fori_loop(0, cnt, body, 0)

    def drain(j, carry):
        pltpu.make_async_copy(ones_v, deg_sh.at[didx_v.at[0, 0]], sem).wait()
        return carry

    lax.fori_loop(0, cnt, drain, 0)
    plsc.subcore_barrier()
    pltpu.sync_copy(deg_sh.at[pl.ds(s * CHUNK, CHUNK)],
                    out_hbm.at[c, pl.ds(s * CHUNK, CHUNK)])


@functools.partial(
    pl.kernel,
    out_type=jax.ShapeDtypeStruct((NC, N_PAD, D), jnp.float32),
    mesh=_mesh,
    scratch_types=[
        pltpu.VMEM((W_MAX, 1, EW), jnp.int32),
        pltpu.VMEM((W_MAX, 1, EW), jnp.int32),
        [pltpu.VMEM((EW, D), jnp.float32)] * 8,
        [pltpu.SemaphoreType.DMA] * 8,
        [pltpu.SemaphoreType.DMA] * 8,
        pltpu.SemaphoreType.DMA,
        pltpu.VMEM_SHARED((N_PAD, D), jnp.float32),
    ],
    compiler_params=_sc_params,
)
def _sc_aggregate(g_hbm, ei_hbm, zeros2_hbm, out_hbm,
                  sidx_v, didx_v, rows, gsem, ssem, zsem, acc_sh):
    c = lax.axis_index("c")
    s = lax.axis_index("s")
    w = c * NS + s
    clamped, shift, cnt = _worker_windows(w)
    d0 = pltpu.async_copy(zeros2_hbm.at[pl.ds(s * CHUNK, CHUNK)],
                          acc_sh.at[pl.ds(s * CHUNK, CHUNK)], zsem)
    d1 = pltpu.async_copy(ei_hbm.at[pl.ds(clamped, W_MAX), pl.ds(0, 1)],
                          sidx_v, gsem[0])
    d2 = pltpu.async_copy(ei_hbm.at[pl.ds(clamped, W_MAX), pl.ds(1, 1)],
                          didx_v, gsem[1])
    d0.wait()
    d1.wait()
    d2.wait()
    plsc.subcore_barrier()

    def slot(j, b, first, tail):
        bb = (b + 4) % 8

        def work():
            pltpu.make_async_copy(g_hbm.at[sidx_v.at[shift + j, 0]],
                                  rows[b], gsem[b]).wait()
            pltpu.async_copy(rows[b], acc_sh.at[didx_v.at[shift + j, 0]],
                             ssem[b], add=True)

        def refill():
            def drain():
                pltpu.make_async_copy(rows[bb], acc_sh.at[didx_v.at[0, 0]],
                                      ssem[bb]).wait()

            if first:
                pl.when(j + 4 >= 8)(drain)
            else:
                drain()
            pltpu.async_copy(g_hbm.at[sidx_v.at[shift + j + 4, 0]],
                             rows[bb], gsem[bb])

        if tail:
            pl.when(j < cnt)(work)
            pl.when(j + 4 < cnt)(refill)
        else:
            work()
            refill()

    for b in range(4):
        pltpu.async_copy(g_hbm.at[sidx_v.at[shift + b, 0]], rows[b], gsem[b])
    for b in range(8):
        slot(b, b, True, False)

    def body(i, carry):
        for b in range(8):
            slot(8 * i + b, b, False, False)
        return carry

    lax.fori_loop(1, (W_MAX - 7) // 8, body, 0)
    for b in range(8):
        slot(72 + b, b, False, True)
    for b in range(8):
        pltpu.make_async_copy(rows[b], acc_sh.at[didx_v.at[0, 0]],
                              ssem[b]).wait()
    plsc.subcore_barrier()
    pltpu.sync_copy(acc_sh.at[pl.ds(s * CHUNK, CHUNK)],
                    out_hbm.at[c, pl.ds(s * CHUNK, CHUNK)])


def _tc1_body(x_ref, w1s_ref, mt_ref, degp_ref, g1f_ref, dinvf_ref):
    deg80 = degp_ref[0] + degp_ref[1] + 1.0
    dinv80 = lax.rsqrt(deg80)
    parts = [jnp.dot(dinv80, mt_ref[t], preferred_element_type=jnp.float32)
             for t in range(16)]
    dinvf = jnp.stack(parts, axis=1).reshape(F_ROWS, 128)
    dinvf_ref[...] = dinvf
    x3 = x_ref[...].reshape(F_REAL, 8, D_IN)
    hf = jnp.zeros((F_REAL, 128), jnp.float32)
    for k in range(8):
        hf = hf + jnp.dot(x3[:, k, :], w1s_ref[k],
                          preferred_element_type=jnp.float32)
    g1f_ref[:F_REAL, :] = hf * dinvf[:F_REAL, :]
    g1f_ref[F_REAL:, :] = jnp.zeros((F_ROWS - F_REAL, 128), jnp.float32)


def _tc2_body(p_ref, g1f_ref, dinvf_ref, b1f_ref, w2b_ref, g2f_ref):
    dinvf = dinvf_ref[...]
    s1 = p_ref[0] + p_ref[1] + g1f_ref[...]
    a1 = s1 * dinvf + b1f_ref[...][None, :]
    h1 = jnp.maximum(a1, 0.0)
    h2 = jnp.dot(h1, w2b_ref[...], preferred_element_type=jnp.float32)
    g2f_ref[...] = h2 * dinvf


def _tc3_body(p_ref, g2f_ref, dinvf_ref, b2f_ref, mavg_ref, msum_ref,
              out_ref):
    dinvf = dinvf_ref[...]
    s2 = p_ref[0] + p_ref[1] + g2f_ref[...]
    a2f = s2 * dinvf + b2f_ref[...][None, :]
    mf = jnp.dot(a2f, mavg_ref[...], preferred_element_type=jnp.float32)
    ef = jnp.exp(a2f - mf)
    sf = jnp.dot(ef, msum_ref[...], preferred_element_type=jnp.float32)
    out_ref[...] = a2f - mf - jnp.log(sf)


_tc1 = pl.pallas_call(
    _tc1_body,
    out_shape=(jax.ShapeDtypeStruct((F_ROWS, 128), jnp.float32),
               jax.ShapeDtypeStruct((F_ROWS, 128), jnp.float32)))
_tc2 = pl.pallas_call(
    _tc2_body, out_shape=jax.ShapeDtypeStruct((F_ROWS, 128), jnp.float32))
_tc3 = pl.pallas_call(
    _tc3_body, out_shape=jax.ShapeDtypeStruct((F_ROWS, 128), jnp.float32))


def kernel(x, edge_index, W1, b1, W2, b2):
    ei_t = edge_index.reshape(2, W_TOT, EW).transpose(1, 0, 2)

    lanes = jnp.arange(128, dtype=jnp.int32)
    kmask = (lanes[None, None, :] // D == jnp.arange(8, dtype=jnp.int32)[:, None, None])
    W1S = jnp.tile(W1, (1, 8))[None, :, :] * kmask.astype(jnp.float32)
    W2b = jnp.kron(jnp.eye(8, dtype=W2.dtype), W2)
    Mt = (jnp.arange(128, dtype=jnp.int32)[None, :, None]
          == 8 * jnp.arange(16, dtype=jnp.int32)[:, None, None]
          + lanes[None, None, :] // D).astype(jnp.float32)
    seg = (lanes[:, None] // D == lanes[None, :] // D).astype(jnp.float32)
    Mavg = seg / float(D)
    Msum = seg
    b1f = jnp.tile(b1, 8)
    b2f = jnp.tile(b2, 8)

    zeros1 = jnp.zeros((N_PAD,), jnp.float32)
    zeros2 = jnp.zeros((N_PAD, D), jnp.float32)

    degp = _sc_degree(ei_t, zeros1)
    g1f, dinvf = _tc1(x, W1S, Mt, degp.reshape(2, 80, 128))
    p1 = _sc_aggregate(g1f.reshape(N_PAD, D), ei_t, zeros2)
    g2f = _tc2(p1.reshape(NC, F_ROWS, 128), g1f, dinvf, b1f, W2b)
    p2 = _sc_aggregate(g2f.reshape(N_PAD, D), ei_t, zeros2)
    outf = _tc3(p2.reshape(NC, F_ROWS, 128), g2f, dinvf, b2f, Mavg, Msum)
    return outf.reshape(N_PAD, D)[:N_NODES]


# speedup vs baseline: 106.1845x; 1.0338x over previous
import functools

import jax
import jax.numpy as jnp
from jax import lax
from jax.experimental import pallas as pl
from jax.experimental.pallas import tpu as pltpu
from jax.experimental.pallas import tpu_sc as plsc

N_NODES = 10000
D_IN = 128
D = 16

NC = 2
NS = 16
NW = NC * NS

N_PAD = 10240
CHUNK = N_PAD // NS
F_ROWS = N_PAD * D // 128
F_REAL = N_NODES * D // 128

EW = 128
E_EDGES = 320000
W_TOT = E_EDGES // EW
W_BASE = W_TOT // NW
W_EXTRA = W_TOT - W_BASE * NW
W_MAX = W_BASE + 1

_mesh = plsc.VectorSubcoreMesh(core_axis_name="c", subcore_axis_name="s")
_sc_params = pltpu.CompilerParams(use_tc_tiling_on_sc=False)


def _worker_windows(w):
    cnt = jnp.where(w < W_EXTRA, W_BASE + 1, W_BASE)
    base = W_BASE * w + jnp.minimum(w, W_EXTRA)
    clamped = jnp.minimum(base, W_TOT - W_MAX)
    return clamped, base - clamped, cnt


@functools.partial(
    pl.kernel,
    out_type=jax.ShapeDtypeStruct((NC, N_PAD), jnp.float32),
    mesh=_mesh,
    scratch_types=[
        pltpu.VMEM((W_MAX, 1, EW), jnp.int32),
        pltpu.VMEM((EW,), jnp.float32),
        pltpu.VMEM_SHARED((N_PAD,), jnp.float32),
        pltpu.SemaphoreType.DMA,
    ],
    compiler_params=_sc_params,
)
def _sc_degree(ei_hbm, zeros1_hbm, out_hbm, didx_v, ones_v, deg_sh, sem):
    c = lax.axis_index("c")
    s = lax.axis_index("s")
    w = c * NS + s
    clamped, shift, cnt = _worker_windows(w)
    d0 = pltpu.async_copy(zeros1_hbm.at[pl.ds(s * CHUNK, CHUNK)],
                          deg_sh.at[pl.ds(s * CHUNK, CHUNK)], sem)
    d1 = pltpu.async_copy(ei_hbm.at[pl.ds(clamped, W_MAX), pl.ds(1, 1)],
                          didx_v, sem)
    d0.wait()
    d1.wait()
    for i in range(EW // 16):
        ones_v[pl.ds(i * 16, 16)] = jnp.ones((16,), jnp.float32)
    plsc.subcore_barrier()

    def body(j, carry):
        pltpu.async_copy(ones_v, deg_sh.at[didx_v.at[shift + j, 0]], sem,
                         add=True)
        return carry

    lax.fori_loop(0, cnt, body, 0)

    def drain(j, carry):
        pltpu.make_async_copy(ones_v, deg_sh.at[didx_v.at[0, 0]], sem).wait()
        return carry

    lax.fori_loop(0, cnt, drain, 0)
    plsc.subcore_barrier()
    pltpu.sync_copy(deg_sh.at[pl.ds(s * CHUNK, CHUNK)],
                    out_hbm.at[c, pl.ds(s * CHUNK, CHUNK)])


@functools.partial(
    pl.kernel,
    out_type=jax.ShapeDtypeStruct((NC, N_PAD, D), jnp.float32),
    mesh=_mesh,
    scratch_types=[
        pltpu.VMEM((W_MAX, 1, EW), jnp.int32),
        pltpu.VMEM((W_MAX, 1, EW), jnp.int32),
        [pltpu.VMEM((EW, D), jnp.float32)] * 8,
        [pltpu.SemaphoreType.DMA] * 8,
        [pltpu.SemaphoreType.DMA] * 8,
        pltpu.SemaphoreType.DMA,
        pltpu.VMEM_SHARED((N_PAD, D), jnp.float32),
    ],
    compiler_params=_sc_params,
)
def _sc_aggregate(g_hbm, ei_hbm, zeros2_hbm, out_hbm,
                  sidx_v, didx_v, rows, gsem, ssem, zsem, acc_sh):
    c = lax.axis_index("c")
    s = lax.axis_index("s")
    w = c * NS + s
    clamped, shift, cnt = _worker_windows(w)
    d0 = pltpu.async_copy(zeros2_hbm.at[pl.ds(s * CHUNK, CHUNK)],
                          acc_sh.at[pl.ds(s * CHUNK, CHUNK)], zsem)
    d1 = pltpu.async_copy(ei_hbm.at[pl.ds(clamped, W_MAX), pl.ds(0, 1)],
                          sidx_v, gsem[0])
    d2 = pltpu.async_copy(ei_hbm.at[pl.ds(clamped, W_MAX), pl.ds(1, 1)],
                          didx_v, gsem[1])
    d0.wait()
    d1.wait()
    d2.wait()
    plsc.subcore_barrier()

    def slot(j, b, first, tail):
        bb = (b + 4) % 8

        def work():
            pltpu.make_async_copy(g_hbm.at[sidx_v.at[shift + j, 0]],
                                  rows[b], gsem[b]).wait()
            pltpu.async_copy(rows[b], acc_sh.at[didx_v.at[shift + j, 0]],
                             ssem[b], add=True)

        def refill():
            def drain():
                pltpu.make_async_copy(rows[bb], acc_sh.at[didx_v.at[0, 0]],
                                      ssem[bb]).wait()

            if first:
                pl.when(j + 4 >= 8)(drain)
            else:
                drain()
            pltpu.async_copy(g_hbm.at[sidx_v.at[shift + j + 4, 0]],
                             rows[bb], gsem[bb])

        if tail:
            pl.when(j < cnt)(work)
            pl.when(j + 4 < cnt)(refill)
        else:
            work()
            refill()

    for b in range(4):
        pltpu.async_copy(g_hbm.at[sidx_v.at[shift + b, 0]], rows[b], gsem[b])
    for b in range(8):
        slot(b, b, True, False)

    def body(i, carry):
        for b in range(8):
            slot(8 * i + b, b, False, False)
        return carry

    lax.fori_loop(1, (W_MAX - 7) // 8, body, 0)
    for b in range(8):
        slot(72 + b, b, False, True)
    for b in range(8):
        pltpu.make_async_copy(rows[b], acc_sh.at[didx_v.at[0, 0]],
                              ssem[b]).wait()
    plsc.subcore_barrier()
    pltpu.sync_copy(acc_sh.at[pl.ds(s * CHUNK, CHUNK)],
                    out_hbm.at[c, pl.ds(s * CHUNK, CHUNK)])


def _tca_body(x_ref, w1s_ref, hf_ref):
    x3 = x_ref[...].reshape(F_REAL, 8, D_IN)
    hf = jnp.zeros((F_REAL, 128), jnp.float32)
    for k in range(8):
        hf = hf + jnp.dot(x3[:, k, :], w1s_ref[k],
                          preferred_element_type=jnp.float32)
    hf_ref[...] = hf


def _tcb_body(hf_ref, mt_ref, degp_ref, g1f_ref, dinvf_ref):
    deg80 = degp_ref[0] + degp_ref[1] + 1.0
    dinv80 = lax.rsqrt(deg80)
    parts = [jnp.dot(dinv80, mt_ref[t], preferred_element_type=jnp.float32)
             for t in range(16)]
    dinvf = jnp.stack(parts, axis=1).reshape(F_ROWS, 128)
    dinvf_ref[...] = dinvf
    g1f_ref[:F_REAL, :] = hf_ref[...] * dinvf[:F_REAL, :]
    g1f_ref[F_REAL:, :] = jnp.zeros((F_ROWS - F_REAL, 128), jnp.float32)


def _tc2_body(p_ref, g1f_ref, dinvf_ref, b1f_ref, w2b_ref, g2f_ref):
    dinvf = dinvf_ref[...]
    s1 = p_ref[0] + p_ref[1] + g1f_ref[...]
    a1 = s1 * dinvf + b1f_ref[...][None, :]
    h1 = jnp.maximum(a1, 0.0)
    h2 = jnp.dot(h1, w2b_ref[...], preferred_element_type=jnp.float32)
    g2f_ref[...] = h2 * dinvf


def _tc3_body(p_ref, g2f_ref, dinvf_ref, b2f_ref, mavg_ref, msum_ref,
              out_ref):
    dinvf = dinvf_ref[...]
    s2 = p_ref[0] + p_ref[1] + g2f_ref[...]
    a2f = s2 * dinvf + b2f_ref[...][None, :]
    mf = jnp.dot(a2f, mavg_ref[...], preferred_element_type=jnp.float32)
    ef = jnp.exp(a2f - mf)
    sf = jnp.dot(ef, msum_ref[...], preferred_element_type=jnp.float32)
    out_ref[...] = a2f - mf - jnp.log(sf)


_tca = pl.pallas_call(
    _tca_body, out_shape=jax.ShapeDtypeStruct((F_REAL, 128), jnp.float32))
_tcb = pl.pallas_call(
    _tcb_body,
    out_shape=(jax.ShapeDtypeStruct((F_ROWS, 128), jnp.float32),
               jax.ShapeDtypeStruct((F_ROWS, 128), jnp.float32)))
_tc2 = pl.pallas_call(
    _tc2_body, out_shape=jax.ShapeDtypeStruct((F_ROWS, 128), jnp.float32))
_tc3 = pl.pallas_call(
    _tc3_body, out_shape=jax.ShapeDtypeStruct((F_ROWS, 128), jnp.float32))


def kernel(x, edge_index, W1, b1, W2, b2):
    ei_t = edge_index.reshape(2, W_TOT, EW).transpose(1, 0, 2)

    lanes = jnp.arange(128, dtype=jnp.int32)
    kmask = (lanes[None, None, :] // D == jnp.arange(8, dtype=jnp.int32)[:, None, None])
    W1S = jnp.tile(W1, (1, 8))[None, :, :] * kmask.astype(jnp.float32)
    W2b = jnp.kron(jnp.eye(8, dtype=W2.dtype), W2)
    Mt = (jnp.arange(128, dtype=jnp.int32)[None, :, None]
          == 8 * jnp.arange(16, dtype=jnp.int32)[:, None, None]
          + lanes[None, None, :] // D).astype(jnp.float32)
    seg = (lanes[:, None] // D == lanes[None, :] // D).astype(jnp.float32)
    Mavg = seg / float(D)
    Msum = seg
    b1f = jnp.tile(b1, 8)
    b2f = jnp.tile(b2, 8)

    zeros1 = jnp.zeros((N_PAD,), jnp.float32)
    zeros2 = jnp.zeros((N_PAD, D), jnp.float32)

    degp = _sc_degree(ei_t, zeros1)
    hf = _tca(x, W1S)
    g1f, dinvf = _tcb(hf, Mt, degp.reshape(2, 80, 128))
    p1 = _sc_aggregate(g1f.reshape(N_PAD, D), ei_t, zeros2)
    g2f = _tc2(p1.reshape(NC, F_ROWS, 128), g1f, dinvf, b1f, W2b)
    p2 = _sc_aggregate(g2f.reshape(N_PAD, D), ei_t, zeros2)
    outf = _tc3(p2.reshape(NC, F_ROWS, 128), g2f, dinvf, b2f, Mavg, Msum)
    return outf.reshape(N_PAD, D)[:N_NODES]
